# Initial kernel scaffold; baseline (speedup 1.0000x reference)
import jax
import jax.numpy as jnp
from jax.experimental import pallas as pl


def kernel(user_embeddings, social_edge_index, social_edge_weights, W_self0, b_self0, W_neigh0, b_neigh0, bias0, W_self1, b_self1, W_neigh1, b_neigh1, bias1, W_agg, b_agg):
    raise NotImplementedError("write your pallas kernel here")


# speedup vs baseline: 3.6449x; 3.6449x over previous
import functools

import jax
import jax.numpy as jnp
from jax import lax
from jax.experimental import pallas as pl
from jax.experimental.pallas import tpu as pltpu
from jax.experimental.pallas import tpu_sc as plsc

_NC = 2
_NS = 16
_NW = _NC * _NS
_L = 16
_CW = 16
_CB = 80
_CK = 16
_NH = 2


def _row_chunk_loop(s, nrch, body_fn):
    kmax = -(-nrch // _NS)

    def step(k, carry):
        g = s + k * _NS

        @pl.when(g < nrch)
        def _():
            body_fn(g * _CK)
        return carry
    lax.fori_loop(0, kmax, step, 0)


@functools.lru_cache(maxsize=None)
def _make_sc_pass(N, D, E):
    DH = D // _NH
    EPW = E // _NW
    NCHUNK = EPW // _CB
    NRCH = N // _CK

    mesh = plsc.VectorSubcoreMesh(core_axis_name="c", subcore_axis_name="s")

    scratch = (
        pltpu.VMEM((NCHUNK, _CB), jnp.int32),
        pltpu.VMEM((NCHUNK, _CB), jnp.int32),
        pltpu.VMEM((NCHUNK, _CB), jnp.int32),
        pltpu.VMEM((EPW,), jnp.float32),
        pltpu.VMEM((_CB, DH), jnp.float32),
        pltpu.VMEM((_CK, DH), jnp.float32),
        pltpu.VMEM_SHARED((N, DH), jnp.float32),
        pltpu.SemaphoreType.DMA,
    )

    def body(xh_hbm, src_hbm, dst_hbm, w_hbm, part_hbm,
             src_v, dst_v, gidx_v, w_v, rows_v, zrows_v, acc, sem):
        c = lax.axis_index("c")
        s = lax.axis_index("s")
        wid = c * _NS + s

        pltpu.sync_copy(src_hbm.at[wid], src_v)
        pltpu.sync_copy(dst_hbm.at[wid], dst_v)
        pltpu.sync_copy(w_hbm.at[pl.ds(wid * EPW, EPW)], w_v)

        def zfill2(r, carry):
            for k in range(DH // _L):
                zrows_v[r, pl.ds(k * _L, _L)] = jnp.zeros((_L,), jnp.float32)
            return carry
        lax.fori_loop(0, _CK, zfill2, 0)

        for h in range(_NH):
            def gfill(j, carry):
                for t in range(_CB // _L):
                    sl = pl.ds(t * _L, _L)
                    gidx_v[j, sl] = src_v[j, sl] * _NH + h
                return carry
            lax.fori_loop(0, NCHUNK, gfill, 0)

            _row_chunk_loop(s, NRCH, lambda r0: pltpu.sync_copy(
                zrows_v, acc.at[pl.ds(r0, _CK)]))
            plsc.subcore_barrier()

            def chunk(j, carry):
                pltpu.async_copy(xh_hbm.at[gidx_v.at[j]], rows_v, sem).wait()

                def scale16(t, inner):
                    wv = w_v[pl.ds(pl.multiple_of(j * _CB + t * _L, _L), _L)]
                    for e in range(_L):
                        wb = wv.at[jnp.full((_L,), e, jnp.int32)].get(
                            mode="promise_in_bounds")
                        row = t * _L + e
                        for k in range(DH // _L):
                            rows_v[row, pl.ds(k * _L, _L)] = (
                                rows_v[row, pl.ds(k * _L, _L)] * wb)
                    return inner
                lax.fori_loop(0, _CB // _L, scale16, 0)

                pltpu.sync_copy(rows_v, acc.at[dst_v.at[j]], add=True)
                return carry
            lax.fori_loop(0, NCHUNK, chunk, 0)

            plsc.subcore_barrier()
            _row_chunk_loop(s, NRCH, lambda r0: pltpu.sync_copy(
                acc.at[pl.ds(r0, _CK)], part_hbm.at[c, h, pl.ds(r0, _CK)]))
            if h + 1 < _NH:
                plsc.subcore_barrier()

    return pl.kernel(
        body,
        out_type=jax.ShapeDtypeStruct((_NC, _NH, N, D // _NH), jnp.float32),
        mesh=mesh,
        scratch_types=scratch,
        compiler_params=pltpu.CompilerParams(use_tc_tiling_on_sc=False),
    )


@functools.lru_cache(maxsize=None)
def _make_sc_counts(N, E):
    EPW = E // _NW
    NCHUNK = EPW // _CB
    NRCH = N // _CK

    mesh = plsc.VectorSubcoreMesh(core_axis_name="c", subcore_axis_name="s")

    scratch = (
        pltpu.VMEM((NCHUNK, _CB), jnp.int32),
        pltpu.VMEM((_CB, _CW), jnp.float32),
        pltpu.VMEM((_CK, _CW), jnp.float32),
        pltpu.VMEM_SHARED((N, _CW), jnp.float32),
    )

    def body(dst_hbm, cnt_hbm, dst_v, ones_v, zrows_v, cntacc):
        c = lax.axis_index("c")
        s = lax.axis_index("s")
        wid = c * _NS + s

        pltpu.sync_copy(dst_hbm.at[wid], dst_v)

        def zfill(r, carry):
            zrows_v[r] = jnp.zeros((_CW,), jnp.float32)
            return carry
        lax.fori_loop(0, _CK, zfill, 0)

        def ofill(e, carry):
            ones_v[e] = jnp.ones((_CW,), jnp.float32)
            return carry
        lax.fori_loop(0, _CB, ofill, 0)

        _row_chunk_loop(s, NRCH, lambda r0: pltpu.sync_copy(
            zrows_v, cntacc.at[pl.ds(r0, _CK)]))
        plsc.subcore_barrier()

        def chunk(j, carry):
            pltpu.sync_copy(ones_v, cntacc.at[dst_v.at[j]], add=True)
            return carry
        lax.fori_loop(0, NCHUNK, chunk, 0)

        plsc.subcore_barrier()
        _row_chunk_loop(s, NRCH, lambda r0: pltpu.sync_copy(
            cntacc.at[pl.ds(r0, _CK)], cnt_hbm.at[c, pl.ds(r0, _CK)]))

    return pl.kernel(
        body,
        out_type=jax.ShapeDtypeStruct((_NC, N, _CW), jnp.float32),
        mesh=mesh,
        scratch_types=scratch,
        compiler_params=pltpu.CompilerParams(use_tc_tiling_on_sc=False),
    )


@functools.lru_cache(maxsize=None)
def _make_tc_layer(N, D, final):
    BN = 1000
    DH = D // _NH
    grid = (N // BN,)

    row_spec = pl.BlockSpec((BN, D), lambda i: (i, 0))
    half_spec = pl.BlockSpec((BN, DH), lambda i: (i, 0))
    cnt_spec = pl.BlockSpec((BN, _CW), lambda i: (i, 0))
    mat_spec = pl.BlockSpec((D, D), lambda i: (0, 0))
    vec_spec = pl.BlockSpec((1, D), lambda i: (0, 0))

    def body(x_ref, p00_ref, p01_ref, p10_ref, p11_ref, c0_ref, c1_ref,
             ws_ref, wn_ref, b_ref, *rest):
        if final:
            wa_ref, ba_ref, out_ref = rest
        else:
            (out_ref,) = rest
        x = x_ref[...]
        agg = jnp.concatenate(
            [p00_ref[...] + p10_ref[...], p01_ref[...] + p11_ref[...]],
            axis=1) + x
        cnt = c0_ref[:, 0:1] + c1_ref[:, 0:1] + 1.0
        agg = agg / cnt
        h = (jnp.dot(x, ws_ref[...], preferred_element_type=jnp.float32)
             + jnp.dot(agg, wn_ref[...], preferred_element_type=jnp.float32)
             + b_ref[...])
        h = jnp.where(h >= 0, h, 0.2 * h)
        if final:
            h = (jnp.dot(h, wa_ref[...], preferred_element_type=jnp.float32)
                 + ba_ref[...])
        out_ref[...] = h

    in_specs = [row_spec, half_spec, half_spec, half_spec, half_spec,
                cnt_spec, cnt_spec, mat_spec, mat_spec, vec_spec]
    if final:
        in_specs += [mat_spec, vec_spec]

    return pl.pallas_call(
        body,
        grid=grid,
        in_specs=in_specs,
        out_specs=row_spec,
        out_shape=jax.ShapeDtypeStruct((N, D), jnp.float32),
    )


def kernel(user_embeddings, social_edge_index, social_edge_weights,
           W_self0, b_self0, W_neigh0, b_neigh0, bias0,
           W_self1, b_self1, W_neigh1, b_neigh1, bias1,
           W_agg, b_agg):
    x = user_embeddings
    N, D = x.shape
    E = social_edge_weights.shape[0]
    EPW = E // _NW
    NCHUNK = EPW // _CB
    DH = D // _NH

    src = social_edge_index[0].reshape(_NW, NCHUNK, _CB)
    dst = social_edge_index[1].reshape(_NW, NCHUNK, _CB)
    w = social_edge_weights

    sc_pass = _make_sc_pass(N, D, E)
    sc_counts = _make_sc_counts(N, E)
    tc0 = _make_tc_layer(N, D, False)
    tc1 = _make_tc_layer(N, D, True)

    b0 = (b_self0 + b_neigh0 + bias0).reshape(1, D)
    b1 = (b_self1 + b_neigh1 + bias1).reshape(1, D)

    cnt = sc_counts(dst)
    part1 = sc_pass(x.reshape(_NH * N, DH), src, dst, w)
    h1 = tc0(x, part1[0, 0], part1[0, 1], part1[1, 0], part1[1, 1],
             cnt[0], cnt[1], W_self0, W_neigh0, b0)
    part2 = sc_pass(h1.reshape(_NH * N, DH), src, dst, w)
    out = tc1(h1, part2[0, 0], part2[0, 1], part2[1, 0], part2[1, 1],
              cnt[0], cnt[1], W_self1, W_neigh1, b1,
              W_agg, b_agg.reshape(1, D))
    return out
